# baseline (speedup 1.0000x reference)
import functools

import jax
import jax.numpy as jnp
from jax import lax
from jax.experimental import pallas as pl
from jax.experimental.pallas import tpu as pltpu
from jax.experimental.pallas import tpu_sc as plsc

N = 10000
D = 128
E = 320000
NEG_SLOPE = 0.2

NC = 2
NS = 16
NT = NC * NS
CW = 128
HW = CW // 2
CHUNKS = 80
E_PAD = NT * CHUNKS * CW
NPAD = 10240
STRIPE = NPAD // NS
PAD_IDX = N
NEG_BIG = -1e30


def _prep(h_ref, w_ref, b_ref, asrc_ref, adst_ref,
          hp_ref, a1_ref, a2_ref, m_ref):
    hp = jnp.dot(h_ref[...], w_ref[...], preferred_element_type=jnp.float32)
    hp = hp + b_ref[0][None, :]
    hp_full = jnp.concatenate(
        [hp, jnp.zeros((NPAD - N, D), jnp.float32)], axis=0)
    hp_ref[...] = hp_full
    a1 = jnp.dot(hp_full, asrc_ref[0])
    a2 = jnp.dot(hp_full, adst_ref[0])
    mask = lax.broadcasted_iota(jnp.int32, (NPAD,), 0) < N
    a1m = jnp.where(mask, a1, NEG_BIG)
    a2m = jnp.where(mask, a2, NEG_BIG)
    a1_ref[...] = a1m.reshape(1, NPAD)
    a2_ref[...] = a2m.reshape(1, NPAD)
    m = jnp.maximum(jnp.max(a1m) + jnp.max(a2m), 0.0)
    m_ref[...] = jnp.full((1, 128), m, jnp.float32)


def _edge_body(rows_hbm, cols_hbm, hp_hbm, a1_hbm, a2_hbm, m_hbm,
               num_out, s_out,
               idx_v, ga1_v, ga2_v, m_v, e_v, hpb_v,
               num_sh, s_sh, a1_sh, a2_sh,
               idx_sem, ga1_sem, ga2_sem, hp_sem):
    c = lax.axis_index("c")
    s = lax.axis_index("s")
    t = c * NS + s

    pltpu.sync_copy(m_hbm.at[0, pl.ds(0, 16)], m_v)

    @pl.when(s == 0)
    def _():
        pltpu.sync_copy(a1_hbm.at[0], a1_sh)
        pltpu.sync_copy(a2_hbm.at[0], a2_sh)

    def _zrow(i, carry):
        for k in range(D // 16):
            hpb_v[0, i, pl.ds(k * 16, 16)] = jnp.zeros((16,), jnp.float32)
        return carry
    lax.fori_loop(0, CW, _zrow, 0)
    for k in range(CW // 16):
        e_v[pl.ds(k * 16, 16)] = jnp.zeros((16,), jnp.float32)
    base = s * STRIPE
    for off in range(0, STRIPE, CW):
        pltpu.sync_copy(hpb_v.at[0], num_sh.at[pl.ds(base + off, CW)])
        pltpu.sync_copy(e_v, s_sh.at[pl.ds(base + off, CW)])
    plsc.subcore_barrier()

    mvec = m_v[...]

    def _idx_start(j, slot):
        pltpu.async_copy(rows_hbm.at[t, j], idx_v.at[slot, 0],
                         idx_sem.at[slot])
        pltpu.async_copy(cols_hbm.at[t, j], idx_v.at[slot, 1],
                         idx_sem.at[slot])

    def _idx_wait(j, slot):
        pltpu.make_async_copy(rows_hbm.at[t, j], idx_v.at[slot, 0],
                              idx_sem.at[slot]).wait()
        pltpu.make_async_copy(cols_hbm.at[t, j], idx_v.at[slot, 1],
                              idx_sem.at[slot]).wait()

    def _gath_start(islot, p):
        pltpu.async_copy(a1_sh.at[idx_v.at[islot, 0]], ga1_v.at[p],
                         ga1_sem.at[p])
        pltpu.async_copy(a2_sh.at[idx_v.at[islot, 1]], ga2_v.at[p],
                         ga2_sem.at[p])
        pltpu.async_copy(hp_hbm.at[idx_v.at[islot, 1]], hpb_v.at[p],
                         hp_sem.at[p])

    def _gath_wait(islot, p):
        pltpu.make_async_copy(a1_sh.at[idx_v.at[islot, 0]], ga1_v.at[p],
                              ga1_sem.at[p]).wait()
        pltpu.make_async_copy(a2_sh.at[idx_v.at[islot, 1]], ga2_v.at[p],
                              ga2_sem.at[p]).wait()
        pltpu.make_async_copy(hp_hbm.at[idx_v.at[islot, 1]], hpb_v.at[p],
                              hp_sem.at[p]).wait()

    _idx_start(0, 0)
    _idx_start(1, 1)
    _idx_wait(0, 0)
    _gath_start(0, 0)

    def _chunk(j, carry):
        p = lax.rem(j, 2)
        q = 1 - p
        islot = lax.rem(j, 3)

        @pl.when(j + 1 < CHUNKS)
        def _():
            is1 = lax.rem(j + 1, 3)
            _idx_wait(j + 1, is1)
            _gath_start(is1, q)

        @pl.when(j + 2 < CHUNKS)
        def _():
            _idx_start(j + 2, lax.rem(j + 2, 3))

        _gath_wait(islot, p)

        ga1 = ga1_v.at[p]
        ga2 = ga2_v.at[p]
        for k in range(CW // 16):
            sl = pl.ds(k * 16, 16)
            x = ga1[sl] + ga2[sl]
            x = jnp.where(x > 0.0, x, NEG_SLOPE * x)
            e_v[sl] = jnp.exp(x - mvec)

        hb = hpb_v.at[p]

        def _wgrp(g, carry2):
            e16 = e_v[pl.ds(g * 16, 16)]
            for ii in range(16):
                es = e16[ii]
                i = g * 16 + ii
                for k in range(D // 16):
                    sl2 = pl.ds(k * 16, 16)
                    hb[i, sl2] = hb[i, sl2] * es
            return carry2
        lax.fori_loop(0, CW // 16, _wgrp, 0)

        pltpu.sync_copy(hb, num_sh.at[idx_v.at[islot, 0]], add=True)
        pltpu.sync_copy(e_v, s_sh.at[idx_v.at[islot, 0]], add=True)
        return carry

    lax.fori_loop(0, CHUNKS, _chunk, 0)
    plsc.subcore_barrier()

    pltpu.sync_copy(num_sh.at[pl.ds(base, STRIPE)],
                    num_out.at[c, pl.ds(base, STRIPE)])
    pltpu.sync_copy(s_sh.at[pl.ds(base, STRIPE)],
                    s_out.at[c, pl.ds(base, STRIPE)])


_edge_kernel = functools.partial(
    pl.kernel,
    out_type=(
        jax.ShapeDtypeStruct((NC, NPAD, D), jnp.float32),
        jax.ShapeDtypeStruct((NC, NPAD), jnp.float32),
    ),
    mesh=plsc.VectorSubcoreMesh(
        core_axis_name="c", subcore_axis_name="s",
        num_cores=NC, num_subcores=NS),
    scratch_types=[
        pltpu.VMEM((3, 2, CW), jnp.int32),
        pltpu.VMEM((2, CW), jnp.float32),
        pltpu.VMEM((2, CW), jnp.float32),
        pltpu.VMEM((16,), jnp.float32),
        pltpu.VMEM((CW,), jnp.float32),
        pltpu.VMEM((2, CW, D), jnp.float32),
        pltpu.VMEM_SHARED((NPAD, D), jnp.float32),
        pltpu.VMEM_SHARED((NPAD,), jnp.float32),
        pltpu.VMEM_SHARED((NPAD,), jnp.float32),
        pltpu.VMEM_SHARED((NPAD,), jnp.float32),
        pltpu.SemaphoreType.DMA((3,)),
        pltpu.SemaphoreType.DMA((2,)),
        pltpu.SemaphoreType.DMA((2,)),
        pltpu.SemaphoreType.DMA((2,)),
    ],
    compiler_params=pltpu.CompilerParams(needs_layout_passes=False),
)(_edge_body)


def _finish(num_ref, s_ref, out_ref):
    n = num_ref[0, :N, :] + num_ref[1, :N, :]
    s = s_ref[0, 0, :N] + s_ref[1, 0, :N]
    out_ref[...] = n / (s + 1e-16)[:, None]


def kernel(edge_index, h, W, b, a_src, a_dst):
    row = edge_index[0]
    col = edge_index[1]
    pad = jnp.full((E_PAD - E,), PAD_IDX, dtype=jnp.int32)
    rows_p = jnp.concatenate([row, pad]).reshape(NT, CHUNKS, CW)
    cols_p = jnp.zeros((NT, CHUNKS, CW), jnp.int32)

    hp_pad, a1, a2, m = pl.pallas_call(
        _prep,
        out_shape=(
            jax.ShapeDtypeStruct((NPAD, D), jnp.float32),
            jax.ShapeDtypeStruct((1, NPAD), jnp.float32),
            jax.ShapeDtypeStruct((1, NPAD), jnp.float32),
            jax.ShapeDtypeStruct((1, 128), jnp.float32),
        ),
    )(h, W, b.reshape(1, D), a_src.reshape(1, D), a_dst.reshape(1, D))

    num_parts, s_parts = _edge_kernel(rows_p, cols_p, hp_pad, a1, a2, m)

    out = pl.pallas_call(
        _finish,
        out_shape=jax.ShapeDtypeStruct((N, D), jnp.float32),
    )(num_parts, s_parts.reshape(NC, 1, NPAD))

    return out


# speedup vs baseline: 23.9153x; 23.9153x over previous
import functools

import jax
import jax.numpy as jnp
from jax import lax
from jax.experimental import pallas as pl
from jax.experimental.pallas import tpu as pltpu
from jax.experimental.pallas import tpu_sc as plsc

N = 10000
D = 128
E = 320000
NEG_SLOPE = 0.2

NC = 2
NS = 16
NT = NC * NS
CW = 128
HW = CW // 2
CHUNKS = 80
E_PAD = NT * CHUNKS * CW
NPAD = 10240
STRIPE = NPAD // NS
PAD_IDX = N
NEG_BIG = -1e30


def _prep(h_ref, w_ref, b_ref, asrc_ref, adst_ref,
          hp_ref, a1_ref, a2_ref, m_ref):
    hp = jnp.dot(h_ref[...], w_ref[...], preferred_element_type=jnp.float32)
    hp = hp + b_ref[0][None, :]
    hp_full = jnp.concatenate(
        [hp, jnp.zeros((NPAD - N, D), jnp.float32)], axis=0)
    hp_ref[...] = hp_full
    a1 = jnp.dot(hp_full, asrc_ref[0])
    a2 = jnp.dot(hp_full, adst_ref[0])
    mask = lax.broadcasted_iota(jnp.int32, (NPAD,), 0) < N
    a1m = jnp.where(mask, a1, NEG_BIG)
    a2m = jnp.where(mask, a2, NEG_BIG)
    a1_ref[...] = a1m.reshape(1, NPAD)
    a2_ref[...] = a2m.reshape(1, NPAD)
    m = jnp.maximum(jnp.max(a1m) + jnp.max(a2m), 0.0)
    m_ref[...] = jnp.full((1, 128), m, jnp.float32)


def _edge_body(rows_hbm, cols_hbm, hp_hbm, a1_hbm, a2_hbm, m_hbm,
               num_out, s_out,
               idx_v, ga1_v, ga2_v, m_v, e_v, hpb_v,
               num_sh, s_sh, a1_sh, a2_sh,
               idx_sem, ga1_sem, ga2_sem, hp_sem, ns_sem, es_sem):
    c = lax.axis_index("c")
    s = lax.axis_index("s")
    t = c * NS + s

    pltpu.sync_copy(m_hbm.at[0, pl.ds(0, 16)], m_v)

    @pl.when(s == 0)
    def _():
        pltpu.sync_copy(a1_hbm.at[0], a1_sh)
        pltpu.sync_copy(a2_hbm.at[0], a2_sh)

    def _zrow(i, carry):
        for k in range(D // 16):
            hpb_v[0, i, pl.ds(k * 16, 16)] = jnp.zeros((16,), jnp.float32)
        return carry
    lax.fori_loop(0, CW, _zrow, 0)
    for k in range(CW // 16):
        e_v[0, pl.ds(k * 16, 16)] = jnp.zeros((16,), jnp.float32)
    base = s * STRIPE
    for off in range(0, STRIPE, CW):
        pltpu.sync_copy(hpb_v.at[0], num_sh.at[pl.ds(base + off, CW)])
        pltpu.sync_copy(e_v.at[0], s_sh.at[pl.ds(base + off, CW)])
    plsc.subcore_barrier()

    mvec = m_v[...]

    def _idx_start(j, slot):
        pltpu.async_copy(rows_hbm.at[t, j], idx_v.at[slot, 0],
                         idx_sem.at[slot])
        pltpu.async_copy(cols_hbm.at[t, j], idx_v.at[slot, 1],
                         idx_sem.at[slot])

    def _idx_wait(j, slot):
        pltpu.make_async_copy(rows_hbm.at[t, j], idx_v.at[slot, 0],
                              idx_sem.at[slot]).wait()
        pltpu.make_async_copy(cols_hbm.at[t, j], idx_v.at[slot, 1],
                              idx_sem.at[slot]).wait()

    def _gath_start(islot, p):
        pltpu.async_copy(a1_sh.at[idx_v.at[islot, 0]], ga1_v.at[p],
                         ga1_sem.at[p])
        pltpu.async_copy(a2_sh.at[idx_v.at[islot, 1]], ga2_v.at[p],
                         ga2_sem.at[p])
        pltpu.async_copy(hp_hbm.at[idx_v.at[islot, 1]], hpb_v.at[p],
                         hp_sem.at[p])

    def _gath_wait(islot, p):
        pltpu.make_async_copy(a1_sh.at[idx_v.at[islot, 0]], ga1_v.at[p],
                              ga1_sem.at[p]).wait()
        pltpu.make_async_copy(a2_sh.at[idx_v.at[islot, 1]], ga2_v.at[p],
                              ga2_sem.at[p]).wait()
        pltpu.make_async_copy(hp_hbm.at[idx_v.at[islot, 1]], hpb_v.at[p],
                              hp_sem.at[p]).wait()

    _idx_start(0, 0)
    _idx_start(1, 1)
    _idx_wait(0, 0)
    _gath_start(0, 0)

    def _chunk(j, carry):
        p = lax.rem(j, 2)
        q = 1 - p
        islot = lax.rem(j, 3)

        @pl.when(j >= 1)
        def _():
            pltpu.make_async_copy(hpb_v.at[q], num_sh.at[idx_v.at[0, 0]],
                                  ns_sem.at[q]).wait()
            pltpu.make_async_copy(e_v.at[q], s_sh.at[idx_v.at[0, 0]],
                                  es_sem.at[q]).wait()

        @pl.when(j + 1 < CHUNKS)
        def _():
            is1 = lax.rem(j + 1, 3)
            _idx_wait(j + 1, is1)
            _gath_start(is1, q)

        @pl.when(j + 2 < CHUNKS)
        def _():
            _idx_start(j + 2, lax.rem(j + 2, 3))

        _gath_wait(islot, p)

        ga1 = ga1_v.at[p]
        ga2 = ga2_v.at[p]
        ev = e_v.at[p]
        for k in range(CW // 16):
            sl = pl.ds(k * 16, 16)
            x = ga1[sl] + ga2[sl]
            x = jnp.where(x > 0.0, x, NEG_SLOPE * x)
            ev[sl] = jnp.exp(x - mvec)

        hb = hpb_v.at[p]

        def _wgrp(g, carry2):
            e16 = ev[pl.ds(g * 16, 16)]
            for ii in range(16):
                es = e16[ii]
                i = g * 16 + ii
                for k in range(D // 16):
                    sl2 = pl.ds(k * 16, 16)
                    hb[i, sl2] = hb[i, sl2] * es
            return carry2
        lax.fori_loop(0, CW // 16, _wgrp, 0)

        pltpu.async_copy(hb, num_sh.at[idx_v.at[islot, 0]], ns_sem.at[p],
                         add=True)
        pltpu.async_copy(ev, s_sh.at[idx_v.at[islot, 0]], es_sem.at[p],
                         add=True)
        return carry

    lax.fori_loop(0, CHUNKS, _chunk, 0)

    last = (CHUNKS - 1) % 2
    pltpu.make_async_copy(hpb_v.at[last], num_sh.at[idx_v.at[0, 0]],
                          ns_sem.at[last]).wait()
    pltpu.make_async_copy(e_v.at[last], s_sh.at[idx_v.at[0, 0]],
                          es_sem.at[last]).wait()
    plsc.subcore_barrier()

    pltpu.sync_copy(num_sh.at[pl.ds(base, STRIPE)],
                    num_out.at[c, pl.ds(base, STRIPE)])
    pltpu.sync_copy(s_sh.at[pl.ds(base, STRIPE)],
                    s_out.at[c, pl.ds(base, STRIPE)])


_edge_kernel = functools.partial(
    pl.kernel,
    out_type=(
        jax.ShapeDtypeStruct((NC, NPAD, D), jnp.float32),
        jax.ShapeDtypeStruct((NC, NPAD), jnp.float32),
    ),
    mesh=plsc.VectorSubcoreMesh(
        core_axis_name="c", subcore_axis_name="s",
        num_cores=NC, num_subcores=NS),
    scratch_types=[
        pltpu.VMEM((3, 2, CW), jnp.int32),
        pltpu.VMEM((2, CW), jnp.float32),
        pltpu.VMEM((2, CW), jnp.float32),
        pltpu.VMEM((16,), jnp.float32),
        pltpu.VMEM((2, CW), jnp.float32),
        pltpu.VMEM((2, CW, D), jnp.float32),
        pltpu.VMEM_SHARED((NPAD, D), jnp.float32),
        pltpu.VMEM_SHARED((NPAD,), jnp.float32),
        pltpu.VMEM_SHARED((NPAD,), jnp.float32),
        pltpu.VMEM_SHARED((NPAD,), jnp.float32),
        pltpu.SemaphoreType.DMA((3,)),
        pltpu.SemaphoreType.DMA((2,)),
        pltpu.SemaphoreType.DMA((2,)),
        pltpu.SemaphoreType.DMA((2,)),
        pltpu.SemaphoreType.DMA((2,)),
        pltpu.SemaphoreType.DMA((2,)),
    ],
    compiler_params=pltpu.CompilerParams(needs_layout_passes=False),
)(_edge_body)


def _finish(num_ref, s_ref, out_ref):
    n = num_ref[0, :N, :] + num_ref[1, :N, :]
    s = s_ref[0, 0, :N] + s_ref[1, 0, :N]
    out_ref[...] = n / (s + 1e-16)[:, None]


def kernel(edge_index, h, W, b, a_src, a_dst):
    row = edge_index[0]
    col = edge_index[1]
    pad = jnp.full((E_PAD - E,), PAD_IDX, dtype=jnp.int32)
    rows_p = jnp.concatenate([row, pad]).reshape(NT, CHUNKS, CW)
    cols_p = jnp.concatenate([col, pad]).reshape(NT, CHUNKS, CW)

    hp_pad, a1, a2, m = pl.pallas_call(
        _prep,
        out_shape=(
            jax.ShapeDtypeStruct((NPAD, D), jnp.float32),
            jax.ShapeDtypeStruct((1, NPAD), jnp.float32),
            jax.ShapeDtypeStruct((1, NPAD), jnp.float32),
            jax.ShapeDtypeStruct((1, 128), jnp.float32),
        ),
    )(h, W, b.reshape(1, D), a_src.reshape(1, D), a_dst.reshape(1, D))

    num_parts, s_parts = _edge_kernel(rows_p, cols_p, hp_pad, a1, a2, m)

    out = pl.pallas_call(
        _finish,
        out_shape=jax.ShapeDtypeStruct((N, D), jnp.float32),
    )(num_parts, s_parts.reshape(NC, 1, NPAD))

    return out
